# baseline (device time: 49166 ns/iter reference)
import jax
import jax.numpy as jnp
from jax import lax
from jax.experimental import pallas as pl
from jax.experimental.pallas import tpu as pltpu

N_DEV = 4
E_LOCAL = 4


def kernel(x, router_W, route_idx, expert_W):
    n_tok, d_model = x.shape
    n_exp_total = router_W.shape[1]
    d_out = expert_W.shape[2]

    def body(x_ref, rw_ref, idx_ref, ew_ref, out_ref, comm_ref,
             send_sems, recv_sems):
        my_i = lax.axis_index("i")
        left = lax.rem(my_i + (N_DEV - 1), N_DEV)
        right = lax.rem(my_i + 1, N_DEV)

        barrier_sem = pltpu.get_barrier_semaphore()
        for nbr in (left, right):
            pl.semaphore_signal(
                barrier_sem, inc=1,
                device_id=(nbr,), device_id_type=pl.DeviceIdType.MESH,
            )
        pl.semaphore_wait(barrier_sem, 2)

        xv = x_ref[:, :]
        scores = jnp.dot(xv, rw_ref[:, :], preferred_element_type=jnp.float32)
        m = jnp.max(scores, axis=-1, keepdims=True)
        p = jnp.exp(scores - m)
        p = p / jnp.sum(p, axis=-1, keepdims=True)

        e0 = idx_ref[:, 0:1]
        e1 = idx_ref[:, 1:2]
        iota = lax.broadcasted_iota(jnp.int32, (n_tok, n_exp_total), 1)
        g0 = jnp.sum(jnp.where(iota == e0, p, 0.0), axis=-1, keepdims=True)
        g1 = jnp.sum(jnp.where(iota == e1, p, 0.0), axis=-1, keepdims=True)
        gs = g0 + g1

        acc = jnp.zeros((n_tok, d_out), jnp.float32)
        for e_loc in range(E_LOCAL):
            e_glob = my_i * E_LOCAL + e_loc
            w = (jnp.where(e0 == e_glob, g0 / gs, 0.0)
                 + jnp.where(e1 == e_glob, g1 / gs, 0.0))
            acc = acc + jnp.dot(xv * w, ew_ref[e_loc],
                                preferred_element_type=jnp.float32)

        out_ref[:, :] = acc
        comm_ref[0, :, :] = acc

        for h in range(N_DEV - 1):
            send_slot = h % 2
            recv_slot = (h + 1) % 2
            rdma = pltpu.make_async_remote_copy(
                src_ref=comm_ref.at[send_slot],
                dst_ref=comm_ref.at[recv_slot],
                send_sem=send_sems.at[h],
                recv_sem=recv_sems.at[h],
                device_id=(right,),
                device_id_type=pl.DeviceIdType.MESH,
            )
            rdma.start()
            rdma.wait()
            out_ref[:, :] = out_ref[:, :] + comm_ref[recv_slot, :, :]

    return pl.pallas_call(
        body,
        out_shape=jax.ShapeDtypeStruct((n_tok, d_out), jnp.float32),
        in_specs=[
            pl.BlockSpec(memory_space=pltpu.VMEM),
            pl.BlockSpec(memory_space=pltpu.VMEM),
            pl.BlockSpec(memory_space=pltpu.VMEM),
            pl.BlockSpec(memory_space=pltpu.VMEM),
        ],
        out_specs=pl.BlockSpec(memory_space=pltpu.VMEM),
        scratch_shapes=[
            pltpu.VMEM((2, n_tok, d_out), jnp.float32),
            pltpu.SemaphoreType.DMA((N_DEV - 1,)),
            pltpu.SemaphoreType.DMA((N_DEV - 1,)),
        ],
        compiler_params=pltpu.CompilerParams(collective_id=0),
    )(x, router_W, route_idx, expert_W)


# device time: 32120 ns/iter; 1.5307x vs baseline; 1.5307x over previous
import jax
import jax.numpy as jnp
from jax import lax
from jax.experimental import pallas as pl
from jax.experimental.pallas import tpu as pltpu

N_DEV = 4
E_LOCAL = 4


def kernel(x, router_W, route_idx, expert_W):
    n_tok, d_model = x.shape
    n_exp_total = router_W.shape[1]
    d_out = expert_W.shape[2]
    half = n_tok // 2

    def body(x_ref, rw_ref, idx_ref, ew_ref, out_ref,
             cw_comm, ccw_comm, cw_send, cw_recv, ccw_send, ccw_recv):
        my_i = lax.axis_index("i")
        left = lax.rem(my_i + (N_DEV - 1), N_DEV)
        right = lax.rem(my_i + 1, N_DEV)

        barrier_sem = pltpu.get_barrier_semaphore()
        for nbr in (left, right):
            pl.semaphore_signal(
                barrier_sem, inc=1,
                device_id=(nbr,), device_id_type=pl.DeviceIdType.MESH,
            )
        pl.semaphore_wait(barrier_sem, 2)

        xv = x_ref[:, :]
        scores = jnp.dot(xv, rw_ref[:, :], preferred_element_type=jnp.float32)
        m = jnp.max(scores, axis=-1, keepdims=True)
        p = jnp.exp(scores - m)
        p = p / jnp.sum(p, axis=-1, keepdims=True)

        e0 = idx_ref[:, 0:1]
        e1 = idx_ref[:, 1:2]
        iota = lax.broadcasted_iota(jnp.int32, (n_tok, n_exp_total), 1)
        g0 = jnp.sum(jnp.where(iota == e0, p, 0.0), axis=-1, keepdims=True)
        g1 = jnp.sum(jnp.where(iota == e1, p, 0.0), axis=-1, keepdims=True)
        gs = g0 + g1

        ws = []
        for e_loc in range(E_LOCAL):
            e_glob = my_i * E_LOCAL + e_loc
            ws.append(jnp.where(e0 == e_glob, g0 / gs, 0.0)
                      + jnp.where(e1 == e_glob, g1 / gs, 0.0))
        ew_flat = ew_ref[:, :, :].reshape(E_LOCAL * d_model, d_out)

        def partial_rows(lo, hi):
            xw = jnp.concatenate(
                [xv[lo:hi] * w[lo:hi] for w in ws], axis=1)
            return jnp.dot(xw, ew_flat, preferred_element_type=jnp.float32)

        def hop(direction_comm, send_sems, recv_sems, dst, h):
            return pltpu.make_async_remote_copy(
                src_ref=direction_comm.at[h],
                dst_ref=direction_comm.at[h + 1],
                send_sem=send_sems.at[h],
                recv_sem=recv_sems.at[h],
                device_id=(dst,),
                device_id_type=pl.DeviceIdType.MESH,
            )

        acc_top = partial_rows(0, half)
        cw_comm[0, :, :] = acc_top
        cw_rdmas = [hop(cw_comm, cw_send, cw_recv, right, h)
                    for h in range(N_DEV - 1)]
        cw_rdmas[0].start()

        acc_bot = partial_rows(half, n_tok)
        ccw_comm[0, :, :] = acc_bot
        ccw_rdmas = [hop(ccw_comm, ccw_send, ccw_recv, left, h)
                     for h in range(N_DEV - 1)]
        ccw_rdmas[0].start()

        out_ref[0:half, :] = acc_top
        out_ref[half:n_tok, :] = acc_bot

        for h in range(N_DEV - 1):
            cw_rdmas[h].wait_recv()
            ccw_rdmas[h].wait_recv()
            if h + 1 < N_DEV - 1:
                cw_rdmas[h + 1].start()
                ccw_rdmas[h + 1].start()
            out_ref[0:half, :] = out_ref[0:half, :] + cw_comm[h + 1, :, :]
            out_ref[half:n_tok, :] = (out_ref[half:n_tok, :]
                                      + ccw_comm[h + 1, :, :])

        for h in range(N_DEV - 1):
            cw_rdmas[h].wait_send()
            ccw_rdmas[h].wait_send()

    return pl.pallas_call(
        body,
        out_shape=jax.ShapeDtypeStruct((n_tok, d_out), jnp.float32),
        in_specs=[pl.BlockSpec(memory_space=pltpu.VMEM)] * 4,
        out_specs=pl.BlockSpec(memory_space=pltpu.VMEM),
        scratch_shapes=[
            pltpu.VMEM((N_DEV, half, d_out), jnp.float32),
            pltpu.VMEM((N_DEV, half, d_out), jnp.float32),
            pltpu.SemaphoreType.DMA((N_DEV - 1,)),
            pltpu.SemaphoreType.DMA((N_DEV - 1,)),
            pltpu.SemaphoreType.DMA((N_DEV - 1,)),
            pltpu.SemaphoreType.DMA((N_DEV - 1,)),
        ],
        compiler_params=pltpu.CompilerParams(collective_id=0),
    )(x, router_W, route_idx, expert_W)
